# baseline (speedup 1.0000x reference)
import functools

import jax
import jax.numpy as jnp
from jax import lax
from jax.experimental import pallas as pl
from jax.experimental.pallas import tpu as pltpu, tpu_sc as plsc

SCORE_THRESH = 0.05
NMS_THRESH = 0.5
DETS_PER_IMG = 100
BBOX_XFORM_CLIP = 4.135166556742356

N_PAD = 5120


def _stage1_kernel(logits_ref, probs_ref, labels_ref, win_ref, col_ref):
    l = logits_ref[0]
    n, c = l.shape
    m = jnp.max(l, axis=1, keepdims=True)
    e = jnp.exp(l - m)
    s = jnp.sum(e, axis=1, keepdims=True)
    probs = 1.0 / s
    cio = jax.lax.broadcasted_iota(jnp.int32, (n, c), 1).astype(jnp.float32)
    labels = jnp.min(jnp.where(e == 1.0, cio, float(c)), axis=1,
                     keepdims=True)

    b = pl.program_id(0)
    rb = pl.program_id(1)
    rows = (b * pl.num_programs(1) + rb) * n + jax.lax.broadcasted_iota(
        jnp.int32, (n, 1), 0)
    o = rows * (4 * c) + labels.astype(jnp.int32) * 4
    probs_ref[0] = probs
    labels_ref[0] = labels
    win_ref[0] = o // 128
    col_ref[0] = o % 128


def _make_sc_gather(total_rows):
    info = plsc.get_sparse_core_info()
    nw = info.num_cores * info.num_subcores
    per_w = total_rows // nw
    mesh = plsc.VectorSubcoreMesh(core_axis_name="c", subcore_axis_name="s")

    @functools.partial(
        pl.kernel, mesh=mesh,
        out_type=[jax.ShapeDtypeStruct((total_rows,), jnp.float32)] * 4,
        scratch_types=[
            pltpu.VMEM((per_w,), jnp.int32),
            pltpu.VMEM((per_w,), jnp.int32),
            pltpu.VMEM((per_w,), jnp.float32),
            pltpu.VMEM((per_w,), jnp.float32),
            pltpu.VMEM((per_w,), jnp.float32),
            pltpu.VMEM((per_w,), jnp.float32),
            pltpu.SemaphoreType.DMA,
            pltpu.SemaphoreType.DMA,
            pltpu.SemaphoreType.DMA,
            pltpu.SemaphoreType.DMA,
        ],
    )
    def gather_k(table_hbm, o_hbm, o0, o1, o2, o3,
                 o_v, oj_v, v0, v1, v2, v3, s0, s1, s2, s3):
        wid = lax.axis_index("s") * info.num_cores + lax.axis_index("c")
        base = wid * per_w
        pltpu.sync_copy(o_hbm.at[pl.ds(base, per_w)], o_v)
        vs = [v0, v1, v2, v3]
        sems = [s0, s1, s2, s3]
        cps = []
        nchunk = per_w // 16
        for j in range(4):
            if j > 0:
                for t in range(nchunk):
                    sl = pl.ds(16 * t, 16)
                    oj_v[sl] = o_v[sl] + j
                cps.append(pltpu.async_copy(table_hbm.at[oj_v], vs[j],
                                            sems[j]))
                cps[-1].wait()
            else:
                pltpu.async_copy(table_hbm.at[o_v], vs[0], sems[0]).wait()
        for j in range(4):
            pltpu.sync_copy(vs[j], [o0, o1, o2, o3][j].at[pl.ds(base, per_w)])

    return gather_k


def _stage2_kernel(nimg, probs_ref, labels_ref, dx_ref, dy_ref, dw_ref,
                   dh_ref, px1_ref, py1_ref, px2_ref, py2_ref,
                   hrow_ref, wrow_ref, boxes_ref, scores_ref, labout_ref):
    probs = probs_ref[...]
    labels = labels_ref[...]

    px1 = px1_ref[...]
    py1 = py1_ref[...]
    px2 = px2_ref[...]
    py2 = py2_ref[...]
    widths = px2 - px1
    heights = py2 - py1
    ctr_x = px1 + 0.5 * widths
    ctr_y = py1 + 0.5 * heights
    dx = dx_ref[...]
    dy = dy_ref[...]
    dw = jnp.minimum(dw_ref[...], BBOX_XFORM_CLIP)
    dh = jnp.minimum(dh_ref[...], BBOX_XFORM_CLIP)
    pred_ctr_x = dx * widths + ctr_x
    pred_ctr_y = dy * heights + ctr_y
    pred_w = jnp.exp(dw) * widths
    pred_h = jnp.exp(dh) * heights
    x1 = pred_ctr_x - 0.5 * pred_w
    y1 = pred_ctr_y - 0.5 * pred_h
    x2 = pred_ctr_x + 0.5 * pred_w
    y2 = pred_ctr_y + 0.5 * pred_h

    h_img = hrow_ref[...]
    w_img = wrow_ref[...]
    x1 = jnp.clip(x1, 0.0, w_img)
    y1 = jnp.clip(y1, 0.0, h_img)
    x2 = jnp.clip(x2, 0.0, w_img)
    y2 = jnp.clip(y2, 0.0, h_img)

    valid = (labels > 0.0) & (probs > SCORE_THRESH)
    sc0 = jnp.where(valid, probs, -1.0)

    lane128 = jax.lax.broadcasted_iota(jnp.int32, (8, 128), 1).astype(
        jnp.float32)
    row8 = jax.lax.broadcasted_iota(jnp.int32, (8, 128), 0).astype(
        jnp.float32)
    idxf = (jax.lax.broadcasted_iota(jnp.int32, (40, 128), 0) * 128
            + jax.lax.broadcasted_iota(jnp.int32, (40, 128), 1)).astype(
        jnp.float32)

    sls = [slice(128 * b, 128 * (b + 1)) for b in range(nimg)]
    maxcs = []
    xo1s, yo1s, xo2s, yo2s, offs, areass, sc0s = [], [], [], [], [], [], []
    for b in range(nimg):
        sl = sls[b]
        mc = jnp.maximum(
            jnp.maximum(jnp.max(x1[:, sl]), jnp.max(y1[:, sl])),
            jnp.maximum(jnp.max(x2[:, sl]), jnp.max(y2[:, sl])))
        maxcs.append(mc)
        off = labels[:, sl] * (mc + 1.0)
        offs.append(off)
        xo1s.append(x1[:, sl] + off)
        yo1s.append(y1[:, sl] + off)
        xo2s.append(x2[:, sl] + off)
        yo2s.append(y2[:, sl] + off)
        areass.append((xo2s[b] - xo1s[b]) * (yo2s[b] - yo1s[b]))
        sc0s.append(sc0[:, sl])

    rio5 = jax.lax.broadcasted_iota(jnp.int32, (5, 200), 0)
    cio5 = jax.lax.broadcasted_iota(jnp.int32, (5, 200), 1)
    blk5 = (cio5 // 40 == rio5).astype(jnp.float32)
    ones_col = jnp.ones((128, 1), jnp.float32)
    dn = (((1,), (0,)), ((), ()))
    hi = jax.lax.Precision.HIGHEST

    def body(k, carry):
        scs, outs = carry
        kf = k.astype(jnp.float32)
        sc_new = []
        out_new = []
        for b in range(nimg):
            scb = scs[b]
            m = jnp.max(scb)
            ok = m > 0.0
            selidx = jnp.min(jnp.where(scb == m, idxf, 1e9))
            sel = (idxf == selidx) & ok

            masked = jnp.concatenate(
                [jnp.where(sel, xo1s[b], 0.0), jnp.where(sel, yo1s[b], 0.0),
                 jnp.where(sel, xo2s[b], 0.0), jnp.where(sel, yo2s[b], 0.0),
                 jnp.where(sel, offs[b], 0.0)], axis=0)
            t = jax.lax.dot_general(blk5, masked, dn, precision=hi,
                                    preferred_element_type=jnp.float32)
            p5 = jax.lax.dot_general(t, ones_col, dn, precision=hi,
                                     preferred_element_type=jnp.float32)
            bx1 = p5[0:1, :]
            by1 = p5[1:2, :]
            bx2 = p5[2:3, :]
            by2 = p5[3:4, :]
            boff = p5[4:5, :]
            barea = (bx2 - bx1) * (by2 - by1)
            xx1 = jnp.maximum(bx1, xo1s[b])
            yy1 = jnp.maximum(by1, yo1s[b])
            xx2 = jnp.minimum(bx2, xo2s[b])
            yy2 = jnp.minimum(by2, yo2s[b])
            inter = (jnp.maximum(xx2 - xx1, 0.0)
                     * jnp.maximum(yy2 - yy1, 0.0))
            iou = inter / (barea + areass[b] - inter + 1e-9)
            kill = (iou > NMS_THRESH) | sel
            sc_new.append(jnp.where(kill & ok, -1.0, scb))

            lab = jnp.floor(boff / (maxcs[b] + 1.0) + 0.5)
            vals = [bx1 - boff, by1 - boff, bx2 - boff, by2 - boff, m, lab]
            slot = (lane128 == kf) & ok
            ob = outs[b]
            for j, v in enumerate(vals):
                ob = jnp.where(slot & (row8 == float(j)), v, ob)
            out_new.append(ob)
        return tuple(sc_new), tuple(out_new)

    out0 = tuple(jnp.zeros((8, 128), jnp.float32) for _ in range(nimg))
    scs, outs = jax.lax.fori_loop(0, DETS_PER_IMG, body,
                                  (tuple(sc0s), out0))

    for b in range(nimg):
        out = outs[b]
        bt = jnp.transpose(out[0:4, :])
        boxes_ref[b] = bt[0:DETS_PER_IMG, :]
        scores_ref[b] = out[4:5, 0:DETS_PER_IMG]
        labout_ref[b] = out[5:6, 0:DETS_PER_IMG].astype(jnp.int32)


@jax.jit
def kernel(class_logits, bbox_deltas, roi_proposals, resized_image_sizes):
    B, N, C = class_logits.shape

    RB = 1000
    nrb = N // RB
    probs, labels, win, col = pl.pallas_call(
        _stage1_kernel,
        grid=(B, nrb),
        in_specs=[pl.BlockSpec((1, RB, C), lambda b, rb: (b, rb, 0))],
        out_specs=[pl.BlockSpec((1, RB, 1), lambda b, rb: (b, rb, 0))] * 4,
        out_shape=[jax.ShapeDtypeStruct((B, N, 1), jnp.float32)] * 2
        + [jax.ShapeDtypeStruct((B, N, 1), jnp.int32)] * 2,
    )(class_logits)

    total = B * N
    total_pad = 20480
    o_flat = jnp.pad((win * 128 + col).reshape(total),
                     (0, total_pad - total))
    table = bbox_deltas.reshape(B * N * 4 * C)
    d4 = _make_sc_gather(total_pad)(table, o_flat)
    dm = jnp.stack([d[:total] for d in d4], axis=-1).reshape(B, N, 4)

    def prep(a, padval):
        a = a.reshape(B, N)
        a = jnp.pad(a, ((0, 0), (0, N_PAD - N)), constant_values=padval)
        a = a.reshape(B, N_PAD // 128, 128)
        return jnp.transpose(a, (1, 0, 2)).reshape(N_PAD // 128, B * 128)

    ins = [prep(probs, -1.0), prep(labels, 0.0)]
    ins += [prep(dm[:, :, j], 0.0) for j in range(4)]
    ins += [prep(roi_proposals[:, :, j], 0.0) for j in range(4)]
    szf = resized_image_sizes.astype(jnp.float32)
    hrow = jnp.repeat(szf[:, 0], 128).reshape(1, B * 128)
    wrow = jnp.repeat(szf[:, 1], 128).reshape(1, B * 128)

    R = N_PAD // 128
    boxes, scores, labout = pl.pallas_call(
        functools.partial(_stage2_kernel, B),
        in_specs=[pl.BlockSpec((R, B * 128), lambda: (0, 0))] * 10
        + [pl.BlockSpec((1, B * 128), lambda: (0, 0))] * 2,
        out_specs=[
            pl.BlockSpec((B, DETS_PER_IMG, 4), lambda: (0, 0, 0)),
            pl.BlockSpec((B, 1, DETS_PER_IMG), lambda: (0, 0, 0)),
            pl.BlockSpec((B, 1, DETS_PER_IMG), lambda: (0, 0, 0)),
        ],
        out_shape=[
            jax.ShapeDtypeStruct((B, DETS_PER_IMG, 4), jnp.float32),
            jax.ShapeDtypeStruct((B, 1, DETS_PER_IMG), jnp.float32),
            jax.ShapeDtypeStruct((B, 1, DETS_PER_IMG), jnp.int32),
        ],
    )(*ins, hrow, wrow)

    return (boxes, scores.reshape(B, DETS_PER_IMG),
            labout.reshape(B, DETS_PER_IMG))
